# baseline (speedup 1.0000x reference)
import functools

import jax
import jax.numpy as jnp
from jax.experimental import pallas as pl
from jax.experimental.pallas import tpu as pltpu

STATE_DIM = 2048
ACTION_DIM = 512
BATCH = 512
FULL_K = STATE_DIM + ACTION_DIM

TILE = 512
N_TILES = STATE_DIM // TILE
NSLOTS_B = 3

_DNT = (((1,), (1,)), ((), ()))


def _dot_t(x, w):
    return jax.lax.dot_general(
        x, w, _DNT,
        precision=jax.lax.Precision.DEFAULT,
        preferred_element_type=jnp.float32,
    )


def _body(
    state_ref, action_ref, ns_hbm,
    w1_ref, b1_ref, w2_hbm, b2_ref, wf_hbm, bf_ref,
    out_ref,
    xn_ref, h_ref, pn_ref, *scr,
):
    b_slots = scr[:NSLOTS_B]
    b_sems = scr[NSLOTS_B:2 * NSLOTS_B]
    sem_n = scr[2 * NSLOTS_B]

    step = pl.program_id(0)

    def b_copy(i):
        t = i % N_TILES
        src = w2_hbm if i < N_TILES else wf_hbm
        return pltpu.make_async_copy(
            src.at[pl.ds(t * TILE, TILE), :],
            b_slots[i % NSLOTS_B], b_sems[i % NSLOTS_B])

    cp_n = pltpu.make_async_copy(ns_hbm, xn_ref, sem_n)

    @pl.when(step == 0)
    def _prime():
        for i in range(NSLOTS_B):
            b_copy(i).start()
        cp_n.start()

    @pl.when(step < N_TILES)
    def _stage1():
        t = step
        col = pl.ds(t * TILE, TILE)
        w = w1_ref[...]
        acc = _dot_t(state_ref[...], w[:, :STATE_DIM])
        acc += _dot_t(action_ref[...], w[:, STATE_DIM:])
        h_ref[:, col] = jnp.maximum(acc + b1_ref[col][None, :], 0.0)

    @pl.when((step >= N_TILES) & (step < 2 * N_TILES))
    def _stage2():
        i = step - N_TILES
        col = pl.ds(i * TILE, TILE)
        for k in range(N_TILES):
            @pl.when(i == k)
            def _():
                b_copy(k).wait()
                pn_ref[:, col] = (_dot_t(h_ref[...], b_slots[k % NSLOTS_B][...])
                                  + b2_ref[col][None, :])
                if k + NSLOTS_B < 2 * N_TILES:
                    b_copy(k + NSLOTS_B).start()

    @pl.when(step >= 2 * N_TILES)
    def _stage3():
        i = step - 2 * N_TILES
        col = pl.ds(i * TILE, TILE)
        for k in range(N_TILES):
            @pl.when(i == k)
            def _():
                b_copy(N_TILES + k).wait()
                if k == 0:
                    cp_n.wait()
                w = b_slots[(N_TILES + k) % NSLOTS_B][...]
                if N_TILES + k + NSLOTS_B < 2 * N_TILES:
                    b_copy(N_TILES + k + NSLOTS_B).start()
                b = bf_ref[col][None, :]
                fa = jnp.maximum(_dot_t(xn_ref[...], w) + b, 0.0)
                fp = jnp.maximum(_dot_t(pn_ref[...], w) + b, 0.0)
                d = fp - fa
                partial = jnp.sum(d * d).reshape(1, 1)

                @pl.when(step == 2 * N_TILES)
                def _():
                    out_ref[...] = jnp.zeros_like(out_ref)

                out_ref[...] += partial


@functools.partial(jax.jit, static_argnames=())
def kernel(state, action, next_state, W1, b1, W2, b2, Wf, bf):
    vmem = functools.partial(pl.BlockSpec, memory_space=pltpu.MemorySpace.VMEM)
    hbm = pl.BlockSpec(memory_space=pltpu.MemorySpace.HBM)
    sse = pl.pallas_call(
        _body,
        grid=(3 * N_TILES,),
        in_specs=[
            vmem(), vmem(), hbm,
            pl.BlockSpec((TILE, FULL_K),
                         lambda s: (jnp.clip(s, 0, N_TILES - 1), 0)),
            vmem(),
            hbm, vmem(),
            hbm, vmem(),
        ],
        out_specs=pl.BlockSpec(memory_space=pltpu.MemorySpace.VMEM),
        out_shape=jax.ShapeDtypeStruct((1, 1), jnp.float32),
        scratch_shapes=(
            [pltpu.VMEM((BATCH, STATE_DIM), jnp.float32),
             pltpu.VMEM((BATCH, STATE_DIM), jnp.float32),
             pltpu.VMEM((BATCH, STATE_DIM), jnp.float32)]
            + [pltpu.VMEM((TILE, STATE_DIM), jnp.float32)
               for _ in range(NSLOTS_B)]
            + [pltpu.SemaphoreType.DMA for _ in range(NSLOTS_B + 1)]
        ),
    )(state, action, next_state, W1, b1, W2, b2, Wf, bf)

    pred_error = sse[0, 0] / jnp.float32(BATCH * STATE_DIM)
    novelty = jnp.float32(1.0)
    curiosity = pred_error * 0.5 + novelty * 0.5
    return jnp.stack([pred_error, novelty, curiosity])


# speedup vs baseline: 1.1062x; 1.1062x over previous
import functools

import jax
import jax.numpy as jnp
from jax.experimental import pallas as pl
from jax.experimental.pallas import tpu as pltpu

STATE_DIM = 2048
ACTION_DIM = 512
BATCH = 512
FULL_K = STATE_DIM + ACTION_DIM

W1_ROWS = 1024
N_W1 = STATE_DIM // W1_ROWS
BROWS = 1024

_B_TILES = [
    ("w2", 0, 1024),
    ("w2", 1024, 1024),
    ("wf", 0, 1024),
    ("wf", 1024, 512),
    ("wf", 1536, 512),
]
NSLOTS_B = 2

_DNT = (((1,), (1,)), ((), ()))


def _dot_t(x, w):
    return jax.lax.dot_general(
        x, w, _DNT,
        precision=jax.lax.Precision.DEFAULT,
        preferred_element_type=jnp.float32,
    )


def _body(
    state_ref, action_ref, ns_hbm,
    w1_hbm, b1_ref, w2_hbm, b2_ref, wf_hbm, bf_ref,
    out_ref,
    xn_ref, h_ref, pn_ref, *scr,
):
    a_bufs = scr[:N_W1]
    b_slots = scr[N_W1:N_W1 + NSLOTS_B]
    a_sems = scr[N_W1 + NSLOTS_B:2 * N_W1 + NSLOTS_B]
    b_sems = scr[2 * N_W1 + NSLOTS_B:2 * N_W1 + 2 * NSLOTS_B]
    sem_n = scr[2 * N_W1 + 2 * NSLOTS_B]

    def a_copy(t):
        return pltpu.make_async_copy(
            w1_hbm.at[pl.ds(t * W1_ROWS, W1_ROWS), :], a_bufs[t], a_sems[t])

    def b_copy(i):
        kind, r0, rows = _B_TILES[i]
        src = w2_hbm if kind == "w2" else wf_hbm
        slot = i % NSLOTS_B
        return pltpu.make_async_copy(
            src.at[pl.ds(r0, rows), :],
            b_slots[slot].at[pl.ds(0, rows), :], b_sems[slot])

    for t in range(N_W1):
        a_copy(t).start()
    for i in range(NSLOTS_B):
        b_copy(i).start()
    cp_n = pltpu.make_async_copy(ns_hbm, xn_ref, sem_n)
    cp_n.start()

    for t in range(N_W1):
        a_copy(t).wait()
        w = a_bufs[t][...]
        col = pl.ds(t * W1_ROWS, W1_ROWS)
        acc = _dot_t(state_ref[...], w[:, :STATE_DIM])
        acc += _dot_t(action_ref[...], w[:, STATE_DIM:])
        h_ref[:, col] = jnp.maximum(acc + b1_ref[col][None, :], 0.0)

    sse = jnp.zeros((), jnp.float32)
    for i, (kind, r0, rows) in enumerate(_B_TILES):
        b_copy(i).wait()
        if kind == "wf" and r0 == 0:
            cp_n.wait()
        w = b_slots[i % NSLOTS_B][pl.ds(0, rows), :]
        col = pl.ds(r0, rows)
        if i + NSLOTS_B < len(_B_TILES):
            b_copy(i + NSLOTS_B).start()
        if kind == "w2":
            pn_ref[:, col] = _dot_t(h_ref[...], w) + b2_ref[col][None, :]
        else:
            b = bf_ref[col][None, :]
            fa = jnp.maximum(_dot_t(xn_ref[...], w) + b, 0.0)
            fp = jnp.maximum(_dot_t(pn_ref[...], w) + b, 0.0)
            d = fp - fa
            sse += jnp.sum(d * d)

    out_ref[...] = sse.reshape(1, 1)


@functools.partial(jax.jit, static_argnames=())
def kernel(state, action, next_state, W1, b1, W2, b2, Wf, bf):
    vmem = functools.partial(pl.BlockSpec, memory_space=pltpu.MemorySpace.VMEM)
    hbm = pl.BlockSpec(memory_space=pltpu.MemorySpace.HBM)
    sse = pl.pallas_call(
        _body,
        in_specs=[
            vmem(), vmem(), hbm,
            hbm, vmem(),
            hbm, vmem(),
            hbm, vmem(),
        ],
        out_specs=vmem(),
        out_shape=jax.ShapeDtypeStruct((1, 1), jnp.float32),
        scratch_shapes=(
            [pltpu.VMEM((BATCH, STATE_DIM), jnp.float32),
             pltpu.VMEM((BATCH, STATE_DIM), jnp.float32),
             pltpu.VMEM((BATCH, STATE_DIM), jnp.float32)]
            + [pltpu.VMEM((W1_ROWS, FULL_K), jnp.float32)
               for _ in range(N_W1)]
            + [pltpu.VMEM((BROWS, STATE_DIM), jnp.float32)
               for _ in range(NSLOTS_B)]
            + [pltpu.SemaphoreType.DMA
               for _ in range(N_W1 + NSLOTS_B + 1)]
        ),
    )(state, action, next_state, W1, b1, W2, b2, Wf, bf)

    pred_error = sse[0, 0] / jnp.float32(BATCH * STATE_DIM)
    novelty = jnp.float32(1.0)
    curiosity = pred_error * 0.5 + novelty * 0.5
    return jnp.stack([pred_error, novelty, curiosity])
